# Initial kernel scaffold; baseline (speedup 1.0000x reference)
import functools

import jax
import jax.numpy as jnp
from jax import lax
from jax.experimental import pallas as pl
from jax.experimental.pallas import tpu as pltpu
from jax.experimental.pallas import tpu_sc as plsc

N_ROWS = 8192
D = 256
K = 8192
BN = 1024
BK = 2048
N_TILES = N_ROWS // BN
K_TILES = K // BK
BIG_I32 = jnp.int32(2**30)


def _dist_argmin_body(x_ref, cb_ref, idx_ref, loss_ref, acc_ref,
                      runmin_ref, runarg_ref):
    n = pl.program_id(0)
    k = pl.program_id(1)
    xb = x_ref[0]
    cb = cb_ref[...]
    s = lax.dot_general(xb, cb, (((0,), (1,)), ((), ())),
                        preferred_element_type=jnp.float32)
    cn = jnp.sum(cb * cb, axis=1)
    d = cn[None, :] - 2.0 * s
    tmin = jnp.min(d, axis=1)
    iota = lax.broadcasted_iota(jnp.int32, (BN, BK), 1)
    targ = jnp.min(jnp.where(d <= tmin[:, None], iota, BIG_I32),
                   axis=1) + k * BK

    @pl.when(k == 0)
    def _():
        runmin_ref[...] = tmin
        runarg_ref[...] = targ

    @pl.when(k > 0)
    def _():
        pm = runmin_ref[...]
        pa = runarg_ref[...]
        upd = tmin < pm
        runmin_ref[...] = jnp.where(upd, tmin, pm)
        runarg_ref[...] = jnp.where(upd, targ, pa)

    @pl.when(k == K_TILES - 1)
    def _():
        zn = jnp.sum(xb * xb, axis=0)
        idx_ref[...] = runarg_ref[...]
        part = jnp.sum(runmin_ref[...] + zn)

        @pl.when(n == 0)
        def _():
            acc_ref[0, 0] = part

        @pl.when(n > 0)
        def _():
            acc_ref[0, 0] = acc_ref[0, 0] + part

        @pl.when(n == N_TILES - 1)
        def _():
            loss_ref[0, 0] = acc_ref[0, 0] * (1.25 / (N_ROWS * D))


def _dist_argmin(xr, codebook):
    return pl.pallas_call(
        _dist_argmin_body,
        grid=(N_TILES, K_TILES),
        in_specs=[
            pl.BlockSpec((1, D, BN), lambda n, k: (n, 0, 0)),
            pl.BlockSpec((BK, D), lambda n, k: (k, 0)),
        ],
        out_specs=[
            pl.BlockSpec((BN,), lambda n, k: (n,)),
            pl.BlockSpec(memory_space=pltpu.SMEM),
        ],
        out_shape=[
            jax.ShapeDtypeStruct((N_ROWS,), jnp.int32),
            jax.ShapeDtypeStruct((1, 1), jnp.float32),
        ],
        scratch_shapes=[
            pltpu.SMEM((1, 1), jnp.float32),
            pltpu.VMEM((BN,), jnp.float32),
            pltpu.VMEM((BN,), jnp.int32),
        ],
        compiler_params=pltpu.CompilerParams(
            dimension_semantics=("arbitrary", "arbitrary")),
    )(xr, codebook)


def _make_sc_gather():
    info = plsc.get_sparse_core_info()
    nc, ns = info.num_cores, info.num_subcores
    nw = nc * ns
    b_per_w = N_ROWS // nw
    n_chunks = b_per_w // 128
    mesh = plsc.VectorSubcoreMesh(core_axis_name="c", subcore_axis_name="s")

    @functools.partial(
        pl.kernel, mesh=mesh,
        out_type=jax.ShapeDtypeStruct((N_ROWS, D), jnp.float32),
        scratch_types=[
            pltpu.VMEM((n_chunks, 128), jnp.int32),
            pltpu.VMEM((b_per_w, D), jnp.float32),
            pltpu.SemaphoreType.DMA,
        ],
    )
    def gather(idx_hbm, table_hbm, out_hbm, idx_v, rows_v, sem):
        wid = lax.axis_index("s") * nc + lax.axis_index("c")
        base = wid * n_chunks
        pltpu.sync_copy(idx_hbm.at[pl.ds(base, n_chunks)], idx_v)
        handles = [
            pltpu.async_copy(table_hbm.at[idx_v.at[j]],
                             rows_v.at[pl.ds(j * 128, 128)], sem)
            for j in range(n_chunks)
        ]
        for h in handles:
            h.wait()
        pltpu.sync_copy(rows_v, out_hbm.at[pl.ds(wid * b_per_w, b_per_w)])

    return gather


_sc_gather = _make_sc_gather()


def kernel(x, codebook):
    xr = x.reshape(8, D, 1024)
    idx, loss2 = _dist_argmin(xr, codebook)
    quant = _sc_gather(idx.reshape(N_ROWS // 128, 128), codebook)
    out = quant.reshape(8, 32, 32, D).transpose(0, 3, 1, 2)
    return out, loss2[0, 0], idx


# speedup vs baseline: 1.0226x; 1.0226x over previous
import functools

import jax
import jax.numpy as jnp
from jax import lax
from jax.experimental import pallas as pl
from jax.experimental.pallas import tpu as pltpu
from jax.experimental.pallas import tpu_sc as plsc

N_ROWS = 8192
D = 256
K = 8192
BN = 1024
BK = 2048
N_TILES = N_ROWS // BN
K_TILES = K // BK


def _dist_argmin_body(z_ref, cbt_ref, idx_ref, loss_ref, acc_ref,
                      runmin_ref, runarg_ref):
    n = pl.program_id(0)
    k = pl.program_id(1)
    zb = z_ref[...]
    cbt = cbt_ref[...]
    s = lax.dot_general(zb, cbt, (((1,), (0,)), ((), ())),
                        preferred_element_type=jnp.float32)
    cn = jnp.sum(cbt * cbt, axis=0)
    d = cn[None, :] - 2.0 * s
    tmin = jnp.min(d, axis=1)
    iota = lax.broadcasted_iota(jnp.int32, (BN, BK), 1)
    targ = jnp.min(jnp.where(d <= tmin[:, None], iota, 2**30),
                   axis=1) + k * BK

    @pl.when(k == 0)
    def _():
        runmin_ref[...] = tmin
        runarg_ref[...] = targ

    @pl.when(k > 0)
    def _():
        pm = runmin_ref[...]
        pa = runarg_ref[...]
        upd = tmin < pm
        runmin_ref[...] = jnp.where(upd, tmin, pm)
        runarg_ref[...] = jnp.where(upd, targ, pa)

    @pl.when(k == K_TILES - 1)
    def _():
        zn = jnp.sum(zb * zb, axis=1)
        idx_ref[...] = runarg_ref[...]
        part = jnp.sum(runmin_ref[...] + zn)

        @pl.when(n == 0)
        def _():
            acc_ref[0, 0] = part

        @pl.when(n > 0)
        def _():
            acc_ref[0, 0] = acc_ref[0, 0] + part

        @pl.when(n == N_TILES - 1)
        def _():
            loss_ref[0, 0] = acc_ref[0, 0] * (1.25 / (N_ROWS * D))


def _dist_argmin(zf, cbt):
    return pl.pallas_call(
        _dist_argmin_body,
        grid=(N_TILES, K_TILES),
        in_specs=[
            pl.BlockSpec((BN, D), lambda n, k: (n, 0)),
            pl.BlockSpec((D, BK), lambda n, k: (0, k)),
        ],
        out_specs=[
            pl.BlockSpec((BN,), lambda n, k: (n,)),
            pl.BlockSpec(memory_space=pltpu.SMEM),
        ],
        out_shape=[
            jax.ShapeDtypeStruct((N_ROWS,), jnp.int32),
            jax.ShapeDtypeStruct((1, 1), jnp.float32),
        ],
        scratch_shapes=[
            pltpu.SMEM((1, 1), jnp.float32),
            pltpu.VMEM((BN,), jnp.float32),
            pltpu.VMEM((BN,), jnp.int32),
        ],
        compiler_params=pltpu.CompilerParams(
            dimension_semantics=("arbitrary", "arbitrary")),
    )(zf, cbt)


@functools.cache
def _make_sc_gather():
    info = plsc.get_sparse_core_info()
    nc, ns = info.num_cores, info.num_subcores
    nw = nc * ns
    b_per_w = N_ROWS // nw
    n_chunks = b_per_w // 128
    mesh = plsc.VectorSubcoreMesh(core_axis_name="c", subcore_axis_name="s")

    @functools.partial(
        pl.kernel, mesh=mesh,
        out_type=jax.ShapeDtypeStruct((N_ROWS, D), jnp.float32),
        scratch_types=[
            pltpu.VMEM((n_chunks, 128), jnp.int32),
            pltpu.VMEM((b_per_w, D), jnp.float32),
            pltpu.SemaphoreType.DMA,
        ],
    )
    def gather(idx_hbm, table_hbm, out_hbm, idx_v, rows_v, sem):
        wid = lax.axis_index("s") * nc + lax.axis_index("c")
        base = wid * n_chunks
        pltpu.sync_copy(idx_hbm.at[pl.ds(base, n_chunks)], idx_v)
        handles = [
            pltpu.async_copy(table_hbm.at[idx_v.at[j]],
                             rows_v.at[pl.ds(j * 128, 128)], sem)
            for j in range(n_chunks)
        ]
        for h in handles:
            h.wait()
        pltpu.sync_copy(rows_v, out_hbm.at[pl.ds(wid * b_per_w, b_per_w)])

    return gather


def kernel(x, codebook):
    zf = jnp.transpose(x, (0, 2, 3, 1)).reshape(N_ROWS, D)
    idx, loss2 = _dist_argmin(zf, codebook.T)
    quant = _make_sc_gather()(idx.reshape(N_ROWS // 128, 128), codebook)
    out = quant.reshape(8, 32, 32, D).transpose(0, 3, 1, 2)
    return out, loss2[0, 0], idx


# speedup vs baseline: 1.0734x; 1.0496x over previous
import functools

import jax
import jax.numpy as jnp
from jax import lax
from jax.experimental import pallas as pl
from jax.experimental.pallas import tpu as pltpu
from jax.experimental.pallas import tpu_sc as plsc

N_ROWS = 8192
D = 256
K = 8192
BN = 1024
BK = 2048
LANES = 128
N_TILES = N_ROWS // BN
K_TILES = K // BK
N_CHUNKS = BK // LANES


def _dist_argmin_body(z_ref, cbt_ref, zn_ref, cn_ref, idx_ref, loss_ref,
                      acc_ref, rv_ref, ri_ref):
    n = pl.program_id(0)
    k = pl.program_id(1)
    zb = z_ref[...]
    cbt = cbt_ref[...]
    znb = zn_ref[...]
    cnb = cn_ref[...]
    s = lax.dot_general(zb, cbt, (((1,), (0,)), ((), ())),
                        preferred_element_type=jnp.float32)
    t = znb[:, None] + s

    @pl.when(k == 0)
    def _():
        rv_ref[...] = jnp.full((BN, LANES), jnp.inf, jnp.float32)
        ri_ref[...] = jnp.zeros((BN, LANES), jnp.int32)

    lane = lax.broadcasted_iota(jnp.int32, (BN, LANES), 1)
    rv = rv_ref[...]
    ri = ri_ref[...]
    for c in range(N_CHUNKS):
        dj = t[:, c * LANES:(c + 1) * LANES] + cnb[None,
                                                   c * LANES:(c + 1) * LANES]
        upd = dj < rv
        rv = jnp.where(upd, dj, rv)
        ri = jnp.where(upd, lane + (k * BK + c * LANES), ri)
    rv_ref[...] = rv
    ri_ref[...] = ri

    @pl.when(k == K_TILES - 1)
    def _():
        tmin = jnp.min(rv, axis=1)
        idx_ref[...] = jnp.min(
            jnp.where(rv <= tmin[:, None], ri, 2**30), axis=1)
        part = jnp.sum(tmin)

        @pl.when(n == 0)
        def _():
            acc_ref[0, 0] = part

        @pl.when(n > 0)
        def _():
            acc_ref[0, 0] = acc_ref[0, 0] + part

        @pl.when(n == N_TILES - 1)
        def _():
            loss_ref[0, 0] = acc_ref[0, 0] * (1.25 / (N_ROWS * D))


def _dist_argmin(zf, cbt, zn, cn):
    return pl.pallas_call(
        _dist_argmin_body,
        grid=(N_TILES, K_TILES),
        in_specs=[
            pl.BlockSpec((BN, D), lambda n, k: (n, 0)),
            pl.BlockSpec((D, BK), lambda n, k: (0, k)),
            pl.BlockSpec((BN,), lambda n, k: (n,)),
            pl.BlockSpec((BK,), lambda n, k: (k,)),
        ],
        out_specs=[
            pl.BlockSpec((BN,), lambda n, k: (n,)),
            pl.BlockSpec(memory_space=pltpu.SMEM),
        ],
        out_shape=[
            jax.ShapeDtypeStruct((N_ROWS,), jnp.int32),
            jax.ShapeDtypeStruct((1, 1), jnp.float32),
        ],
        scratch_shapes=[
            pltpu.SMEM((1, 1), jnp.float32),
            pltpu.VMEM((BN, LANES), jnp.float32),
            pltpu.VMEM((BN, LANES), jnp.int32),
        ],
        compiler_params=pltpu.CompilerParams(
            dimension_semantics=("arbitrary", "arbitrary")),
    )(zf, cbt, zn, cn)


@functools.cache
def _make_sc_gather():
    info = plsc.get_sparse_core_info()
    nc, ns = info.num_cores, info.num_subcores
    nw = nc * ns
    b_per_w = N_ROWS // nw
    n_chunks = b_per_w // 128
    mesh = plsc.VectorSubcoreMesh(core_axis_name="c", subcore_axis_name="s")

    @functools.partial(
        pl.kernel, mesh=mesh,
        out_type=jax.ShapeDtypeStruct((N_ROWS, D), jnp.float32),
        scratch_types=[
            pltpu.VMEM((n_chunks, 128), jnp.int32),
            pltpu.VMEM((b_per_w, D), jnp.float32),
            pltpu.SemaphoreType.DMA,
        ],
    )
    def gather(idx_hbm, table_hbm, out_hbm, idx_v, rows_v, sem):
        wid = lax.axis_index("s") * nc + lax.axis_index("c")
        base = wid * n_chunks
        pltpu.sync_copy(idx_hbm.at[pl.ds(base, n_chunks)], idx_v)
        handles = [
            pltpu.async_copy(table_hbm.at[idx_v.at[j]],
                             rows_v.at[pl.ds(j * 128, 128)], sem)
            for j in range(n_chunks)
        ]
        for h in handles:
            h.wait()
        pltpu.sync_copy(rows_v, out_hbm.at[pl.ds(wid * b_per_w, b_per_w)])

    return gather


def kernel(x, codebook):
    zf = jnp.transpose(x, (0, 2, 3, 1)).reshape(N_ROWS, D)
    zn = jnp.sum(zf ** 2, axis=1)
    cn = jnp.sum(codebook ** 2, axis=1)
    idx, loss2 = _dist_argmin(zf, codebook.T * -2.0, zn, cn)
    quant = _make_sc_gather()(idx.reshape(N_ROWS // 128, 128), codebook)
    out = quant.reshape(8, 32, 32, D).transpose(0, 3, 1, 2)
    return out, loss2[0, 0], idx
